# Initial kernel scaffold; baseline (speedup 1.0000x reference)
import functools

import jax
import jax.numpy as jnp
from jax import lax
from jax.experimental import pallas as pl
from jax.experimental.pallas import tpu as pltpu
from jax.experimental.pallas import tpu_sc as plsc

N_NODES = 100000
N_EDGES = 6400000
N_STEPS = 5

NC = 2
NS = 16
N_TILES = NC * NS

SLICE = 6272
NP = NS * SLICE
RSLICE = NP // N_TILES

CHUNK_R = 16
CHUNK_C = 128
CHUNK = CHUNK_R * CHUNK_C
EDGES_PER_TILE = -(-N_EDGES // (N_TILES * CHUNK)) * CHUNK
N_CHUNKS = EDGES_PER_TILE // CHUNK
EPAD = EDGES_PER_TILE * N_TILES

_mesh = plsc.VectorSubcoreMesh(core_axis_name="c", subcore_axis_name="s")


def _step_body(parts_in, res_in, brow, src, dst, prb,
               parts_out, res_out,
               x_buf, stage, src_v, dst_v, prb_v, val_v, b_v, acc):
    cid = lax.axis_index("c")
    sid = lax.axis_index("s")
    tid = cid * NS + sid

    pltpu.sync_copy(parts_in.at[0], x_buf)
    for s2 in range(NS):
        pltpu.sync_copy(parts_in.at[1, pl.ds(s2 * SLICE, SLICE)], stage)

        def add_body(j, _):
            off = s2 * SLICE + j * 16
            x_buf[pl.ds(off, 16)] = x_buf[pl.ds(off, 16)] + stage[pl.ds(j * 16, 16)]
            return _

        lax.fori_loop(0, SLICE // 16, add_body, 0)

    zv = jnp.zeros((16,), jnp.float32)

    def zero_body(j, _):
        stage[pl.ds(j * 16, 16)] = zv
        return _

    lax.fori_loop(0, SLICE // 16, zero_body, 0)
    pltpu.sync_copy(stage, acc.at[pl.ds(sid * SLICE, SLICE)])
    plsc.subcore_barrier()

    def chunk_body(g, _):
        pltpu.sync_copy(src.at[tid, g], src_v)
        pltpu.sync_copy(dst.at[tid, g], dst_v)
        pltpu.sync_copy(prb.at[tid, g], prb_v)
        for r in range(CHUNK_R):
            for j in range(CHUNK_C // 16):
                idx = src_v[r, pl.ds(j * 16, 16)]
                v = plsc.load_gather(x_buf, [idx]) * prb_v[r, pl.ds(j * 16, 16)]
                val_v[r, pl.ds(j * 16, 16)] = v
        pltpu.sync_copy(val_v, acc.at[dst_v], add=True)
        return _

    lax.fori_loop(0, N_CHUNKS, chunk_body, 0)
    plsc.subcore_barrier()

    pltpu.sync_copy(acc.at[pl.ds(sid * SLICE, SLICE)], stage)
    pltpu.sync_copy(stage, parts_out.at[cid, pl.ds(sid * SLICE, SLICE)])

    pltpu.sync_copy(brow, b_v)
    base = tid * RSLICE
    pltpu.sync_copy(res_in.at[pl.ds(base, RSLICE)], stage.at[pl.ds(0, RSLICE)])

    def res_body(j, _):
        off = j * 16
        stage[pl.ds(off, 16)] = (stage[pl.ds(off, 16)]
                                 + b_v[...] * x_buf[pl.ds(base + off, 16)])
        return _

    lax.fori_loop(0, RSLICE // 16, res_body, 0)
    pltpu.sync_copy(stage.at[pl.ds(0, RSLICE)], res_out.at[pl.ds(base, RSLICE)])


_step = functools.partial(
    pl.kernel,
    out_type=(jax.ShapeDtypeStruct((NC, NP), jnp.float32),
              jax.ShapeDtypeStruct((NP,), jnp.float32)),
    mesh=_mesh,
    scratch_types=[
        pltpu.VMEM((NP,), jnp.float32),
        pltpu.VMEM((SLICE,), jnp.float32),
        pltpu.VMEM((CHUNK_R, CHUNK_C), jnp.int32),
        pltpu.VMEM((CHUNK_R, CHUNK_C), jnp.int32),
        pltpu.VMEM((CHUNK_R, CHUNK_C), jnp.float32),
        pltpu.VMEM((CHUNK_R, CHUNK_C), jnp.float32),
        pltpu.VMEM((16,), jnp.float32),
        pltpu.VMEM_SHARED((NP,), jnp.float32),
    ],
)(_step_body)


def _final_body(parts, res_in, brow, out, stage0, stage1, b_v):
    cid = lax.axis_index("c")
    sid = lax.axis_index("s")
    tid = cid * NS + sid
    base = tid * RSLICE

    pltpu.sync_copy(brow, b_v)
    pltpu.sync_copy(res_in.at[pl.ds(base, RSLICE)], stage0)
    pltpu.sync_copy(parts.at[0, pl.ds(base, RSLICE)], stage1)

    def add0(j, _):
        off = j * 16
        stage0[pl.ds(off, 16)] = (stage0[pl.ds(off, 16)]
                                  + b_v[...] * stage1[pl.ds(off, 16)])
        return _

    lax.fori_loop(0, RSLICE // 16, add0, 0)
    pltpu.sync_copy(parts.at[1, pl.ds(base, RSLICE)], stage1)
    lax.fori_loop(0, RSLICE // 16, add0, 0)
    pltpu.sync_copy(stage0, out.at[pl.ds(base, RSLICE)])


_final = functools.partial(
    pl.kernel,
    out_type=jax.ShapeDtypeStruct((NP,), jnp.float32),
    mesh=_mesh,
    scratch_types=[
        pltpu.VMEM((RSLICE,), jnp.float32),
        pltpu.VMEM((RSLICE,), jnp.float32),
        pltpu.VMEM((16,), jnp.float32),
    ],
)(_final_body)


def kernel(x, edge_index, edge_probs, weights):
    src = edge_index[0].astype(jnp.int32)
    dst = edge_index[1].astype(jnp.int32)
    probs = edge_probs.astype(jnp.float32)

    pad = EPAD - N_EDGES
    src_p = jnp.pad(src, (0, pad)).reshape(N_TILES, N_CHUNKS, CHUNK_R, CHUNK_C)
    dst_p = jnp.pad(dst, (0, pad)).reshape(N_TILES, N_CHUNKS, CHUNK_R, CHUNK_C)
    prb_p = jnp.pad(probs, (0, pad)).reshape(N_TILES, N_CHUNKS, CHUNK_R, CHUNK_C)

    x_flat = jnp.pad(x[:, 0], (0, NP - N_NODES))
    zeros_np = jnp.zeros((NP,), jnp.float32)
    parts = jnp.stack([x_flat, zeros_np])
    res = zeros_np
    w_rows = jnp.broadcast_to(weights[:, None], (N_STEPS, 16))

    for k in range(1, N_STEPS):
        parts, res = _step(parts, res, w_rows[k - 1], src_p, dst_p, prb_p)
    out = _final(parts, res, w_rows[N_STEPS - 1])
    return out[:N_NODES, None]


# speedup vs baseline: 123.4486x; 123.4486x over previous
import functools

import jax
import jax.numpy as jnp
from jax import lax
from jax.experimental import pallas as pl
from jax.experimental.pallas import tpu as pltpu
from jax.experimental.pallas import tpu_sc as plsc

N_NODES = 100000
N_EDGES = 6400000
N_STEPS = 5

NC = 2
NS = 16
N_TILES = NC * NS

SLICE = 6400
NP = NS * SLICE
RSLICE = NP // N_TILES

CHUNK_R = 16
CHUNK_C = 128
CHUNK = CHUNK_R * CHUNK_C
EDGES_PER_TILE = -(-N_EDGES // (N_TILES * CHUNK)) * CHUNK
N_CHUNKS = EDGES_PER_TILE // CHUNK
EPAD = EDGES_PER_TILE * N_TILES

_mesh = plsc.VectorSubcoreMesh(core_axis_name="c", subcore_axis_name="s")
_params = pltpu.CompilerParams(needs_layout_passes=False)


def _step_body(pa_in, pb_in, res_in, brow, src, dst, prb,
               pa_out, pb_out, res_out,
               x_buf, stage, src_v, dst_v, prb_v, val_v, b_v, acc):
    cid = lax.axis_index("c")
    sid = lax.axis_index("s")
    tid = cid * NS + sid

    pltpu.sync_copy(pa_in, x_buf)
    for s2 in range(NS):
        pltpu.sync_copy(pb_in.at[pl.ds(s2 * SLICE, SLICE)], stage)

        def add_body(j, _):
            off = s2 * SLICE + j * 16
            x_buf[pl.ds(off, 16)] = x_buf[pl.ds(off, 16)] + stage[pl.ds(j * 16, 16)]
            return _

        lax.fori_loop(0, SLICE // 16, add_body, 0)

    zv = jnp.zeros((16,), jnp.float32)

    def zero_body(j, _):
        stage[pl.ds(j * 16, 16)] = zv
        return _

    lax.fori_loop(0, SLICE // 16, zero_body, 0)
    pltpu.sync_copy(stage, acc.at[pl.ds(sid * SLICE, SLICE)])
    plsc.subcore_barrier()

    def chunk_body(g, _):
        pltpu.sync_copy(src.at[tid, g], src_v)
        pltpu.sync_copy(dst.at[tid, g], dst_v)
        pltpu.sync_copy(prb.at[tid, g], prb_v)
        for r in range(CHUNK_R):
            for j in range(CHUNK_C // 16):
                idx = src_v[r, pl.ds(j * 16, 16)]
                v = plsc.load_gather(x_buf, [idx]) * prb_v[r, pl.ds(j * 16, 16)]
                val_v[r, pl.ds(j * 16, 16)] = v
        for r in range(CHUNK_R):
            pltpu.sync_copy(val_v.at[r], acc.at[dst_v.at[r]], add=True)
        return _

    lax.fori_loop(0, N_CHUNKS, chunk_body, 0)
    plsc.subcore_barrier()

    pltpu.sync_copy(acc.at[pl.ds(sid * SLICE, SLICE)], stage)

    @pl.when(cid == 0)
    def _():
        pltpu.sync_copy(stage, pa_out.at[pl.ds(sid * SLICE, SLICE)])

    @pl.when(cid == 1)
    def _():
        pltpu.sync_copy(stage, pb_out.at[pl.ds(sid * SLICE, SLICE)])

    pltpu.sync_copy(brow, b_v)
    base = tid * RSLICE
    pltpu.sync_copy(res_in.at[pl.ds(base, RSLICE)], stage.at[pl.ds(0, RSLICE)])

    def res_body(j, _):
        off = j * 16
        stage[pl.ds(off, 16)] = (stage[pl.ds(off, 16)]
                                 + b_v[...] * x_buf[pl.ds(base + off, 16)])
        return _

    lax.fori_loop(0, RSLICE // 16, res_body, 0)
    pltpu.sync_copy(stage.at[pl.ds(0, RSLICE)], res_out.at[pl.ds(base, RSLICE)])


_step = functools.partial(
    pl.kernel,
    out_type=(jax.ShapeDtypeStruct((NP,), jnp.float32),
              jax.ShapeDtypeStruct((NP,), jnp.float32),
              jax.ShapeDtypeStruct((NP,), jnp.float32)),
    mesh=_mesh,
    scratch_types=[
        pltpu.VMEM((NP,), jnp.float32),
        pltpu.VMEM((SLICE,), jnp.float32),
        pltpu.VMEM((CHUNK_R, CHUNK_C), jnp.int32),
        pltpu.VMEM((CHUNK_R, CHUNK_C), jnp.int32),
        pltpu.VMEM((CHUNK_R, CHUNK_C), jnp.float32),
        pltpu.VMEM((CHUNK_R, CHUNK_C), jnp.float32),
        pltpu.VMEM((16,), jnp.float32),
        pltpu.VMEM_SHARED((NP,), jnp.float32),
    ],
    compiler_params=_params,
)(_step_body)


def _final_body(pa, pb, res_in, brow, out, stage0, stage1, b_v):
    cid = lax.axis_index("c")
    sid = lax.axis_index("s")
    tid = cid * NS + sid
    base = tid * RSLICE

    pltpu.sync_copy(brow, b_v)
    pltpu.sync_copy(res_in.at[pl.ds(base, RSLICE)], stage0)
    pltpu.sync_copy(pa.at[pl.ds(base, RSLICE)], stage1)

    def add0(j, _):
        off = j * 16
        stage0[pl.ds(off, 16)] = (stage0[pl.ds(off, 16)]
                                  + b_v[...] * stage1[pl.ds(off, 16)])
        return _

    lax.fori_loop(0, RSLICE // 16, add0, 0)
    pltpu.sync_copy(pb.at[pl.ds(base, RSLICE)], stage1)
    lax.fori_loop(0, RSLICE // 16, add0, 0)
    pltpu.sync_copy(stage0, out.at[pl.ds(base, RSLICE)])


_final = functools.partial(
    pl.kernel,
    out_type=jax.ShapeDtypeStruct((NP,), jnp.float32),
    mesh=_mesh,
    scratch_types=[
        pltpu.VMEM((RSLICE,), jnp.float32),
        pltpu.VMEM((RSLICE,), jnp.float32),
        pltpu.VMEM((16,), jnp.float32),
    ],
    compiler_params=_params,
)(_final_body)


def kernel(x, edge_index, edge_probs, weights):
    src = edge_index[0].astype(jnp.int32)
    dst = edge_index[1].astype(jnp.int32)
    probs = edge_probs.astype(jnp.float32)

    pad = EPAD - N_EDGES
    src_p = jnp.pad(src, (0, pad)).reshape(N_TILES, N_CHUNKS, CHUNK_R, CHUNK_C)
    dst_p = jnp.pad(dst, (0, pad)).reshape(N_TILES, N_CHUNKS, CHUNK_R, CHUNK_C)
    prb_p = jnp.pad(probs, (0, pad)).reshape(N_TILES, N_CHUNKS, CHUNK_R, CHUNK_C)

    pa = jnp.pad(x[:, 0], (0, NP - N_NODES))
    pb = jnp.zeros((NP,), jnp.float32)
    res = pb
    w_rows = jnp.broadcast_to(weights[:, None], (N_STEPS, 16))

    for k in range(1, N_STEPS):
        pa, pb, res = _step(pa, pb, res, w_rows[k - 1], src_p, dst_p, prb_p)
    out = _final(pa, pb, res, w_rows[N_STEPS - 1])
    return out[:N_NODES, None]


# speedup vs baseline: 375.7796x; 3.0440x over previous
import functools

import jax
import jax.numpy as jnp
from jax import lax
from jax.experimental import pallas as pl
from jax.experimental.pallas import tpu as pltpu
from jax.experimental.pallas import tpu_sc as plsc

N_NODES = 100000
N_EDGES = 6400000
N_STEPS = 5

NC = 2
NS = 16
N_TILES = NC * NS

SLICE = 6400
NP = NS * SLICE

CHUNK_R = 8
CHUNK_C = 128
CHUNK = CHUNK_R * CHUNK_C
EDGES_PER_TILE = -(-N_EDGES // (N_TILES * CHUNK)) * CHUNK
N_CHUNKS = EDGES_PER_TILE // CHUNK
EPAD = EDGES_PER_TILE * N_TILES
NBUF = 3

_PIECES = [(0, 2048), (2048, 2048), (4096, 2048), (6144, 256)]

_mesh = plsc.VectorSubcoreMesh(core_axis_name="c", subcore_axis_name="s")
_params = pltpu.CompilerParams(needs_layout_passes=False)


def _step_body(x_in, src, dst, prb, pa_out, pb_out,
               x_buf, stage,
               src_v0, src_v1, src_v2, dst_v0, dst_v1, dst_v2,
               prb_v0, prb_v1, prb_v2, val_v0, val_v1, val_v2,
               sem_in0, sem_in1, sem_in2, sem_sc, acc):
    cid = lax.axis_index("c")
    sid = lax.axis_index("s")
    tid = cid * NS + sid
    base = sid * SLICE
    sems = [sem_in0, sem_in1, sem_in2]
    src_vs = [src_v0, src_v1, src_v2]
    dst_vs = [dst_v0, dst_v1, dst_v2]
    prb_vs = [prb_v0, prb_v1, prb_v2]
    val_vs = [val_v0, val_v1, val_v2]

    pltpu.sync_copy(x_in, x_buf)

    zv = jnp.zeros((16,), jnp.float32)

    def zero_body(j, _):
        stage[pl.ds(j * 16, 16)] = zv
        return _

    lax.fori_loop(0, 128, zero_body, 0)
    for off, sz in _PIECES:
        pltpu.sync_copy(stage.at[pl.ds(0, sz)], acc.at[pl.ds(base + off, sz)])
    plsc.subcore_barrier()

    def issue(g, b):
        pltpu.async_copy(src.at[tid, g], src_vs[b], sems[b])
        pltpu.async_copy(dst.at[tid, g], dst_vs[b], sems[b])
        pltpu.async_copy(prb.at[tid, g], prb_vs[b], sems[b])

    def wait_in(g, b):
        pltpu.make_async_copy(src.at[tid, g], src_vs[b], sems[b]).wait()
        pltpu.make_async_copy(dst.at[tid, g], dst_vs[b], sems[b]).wait()
        pltpu.make_async_copy(prb.at[tid, g], prb_vs[b], sems[b]).wait()

    def gather(b):
        for r in range(CHUNK_R):
            for j in range(CHUNK_C // 16):
                idx = src_vs[b][r, pl.ds(j * 16, 16)]
                v = (plsc.load_gather(x_buf, [idx])
                     * prb_vs[b][r, pl.ds(j * 16, 16)])
                val_vs[b][r, pl.ds(j * 16, 16)] = v

    def fire(b):
        for r in range(CHUNK_R):
            pltpu.async_copy(val_vs[b].at[r], acc.at[dst_vs[b].at[r]], sem_sc,
                             add=True)

    def drain(b):
        pltpu.make_async_copy(src.at[tid, 0], dst_vs[b], sem_sc).wait()

    issue(0, 0)
    issue(1, 1)

    def loop_body(t, carry):
        for u in range(NBUF):
            i = t * NBUF + u
            wait_in(i, u)
            gather(u)
            if u == 0:
                @pl.when(t >= 1)
                def _():
                    drain(NBUF - 1)
            else:
                drain(u - 1)
            issue(i + 2, (u + 2) % NBUF)
            fire(u)
        return carry

    n_main = (N_CHUNKS - 2) // NBUF
    lax.fori_loop(0, n_main, loop_body, 0)
    for i in range(n_main * NBUF, N_CHUNKS):
        u = i % NBUF
        wait_in(i, u)
        gather(u)
        drain((u - 1) % NBUF)
        if i + 2 < N_CHUNKS:
            issue(i + 2, (i + 2) % NBUF)
        fire(u)
    drain((N_CHUNKS - 1) % NBUF)
    plsc.subcore_barrier()

    for off, sz in _PIECES:
        pltpu.sync_copy(acc.at[pl.ds(base + off, sz)], stage.at[pl.ds(0, sz)])

        @pl.when(cid == 0)
        def _():
            pltpu.sync_copy(stage.at[pl.ds(0, sz)],
                            pa_out.at[pl.ds(base + off, sz)])

        @pl.when(cid == 1)
        def _():
            pltpu.sync_copy(stage.at[pl.ds(0, sz)],
                            pb_out.at[pl.ds(base + off, sz)])


_step = functools.partial(
    pl.kernel,
    out_type=(jax.ShapeDtypeStruct((NP,), jnp.float32),
              jax.ShapeDtypeStruct((NP,), jnp.float32)),
    mesh=_mesh,
    scratch_types=[
        pltpu.VMEM((NP,), jnp.float32),
        pltpu.VMEM((2048,), jnp.float32),
        pltpu.VMEM((CHUNK_R, CHUNK_C), jnp.int32),
        pltpu.VMEM((CHUNK_R, CHUNK_C), jnp.int32),
        pltpu.VMEM((CHUNK_R, CHUNK_C), jnp.int32),
        pltpu.VMEM((CHUNK_R, CHUNK_C), jnp.int32),
        pltpu.VMEM((CHUNK_R, CHUNK_C), jnp.int32),
        pltpu.VMEM((CHUNK_R, CHUNK_C), jnp.int32),
        pltpu.VMEM((CHUNK_R, CHUNK_C), jnp.float32),
        pltpu.VMEM((CHUNK_R, CHUNK_C), jnp.float32),
        pltpu.VMEM((CHUNK_R, CHUNK_C), jnp.float32),
        pltpu.VMEM((CHUNK_R, CHUNK_C), jnp.float32),
        pltpu.VMEM((CHUNK_R, CHUNK_C), jnp.float32),
        pltpu.VMEM((CHUNK_R, CHUNK_C), jnp.float32),
        pltpu.SemaphoreType.DMA,
        pltpu.SemaphoreType.DMA,
        pltpu.SemaphoreType.DMA,
        pltpu.SemaphoreType.DMA,
        pltpu.VMEM_SHARED((NP,), jnp.float32),
    ],
    compiler_params=_params,
)(_step_body)


def _combine_body(w_ref, pa_ref, pb_ref, res_ref, y_out, res_out):
    y = pa_ref[...] + pb_ref[...]
    y_out[...] = y
    res_out[...] = res_ref[...] + w_ref[0] * y


def _combine(w, pa, pb, res):
    y2, r2 = pl.pallas_call(
        _combine_body,
        out_shape=(jax.ShapeDtypeStruct((NP // 128, 128), jnp.float32),
                   jax.ShapeDtypeStruct((NP // 128, 128), jnp.float32)),
        in_specs=[
            pl.BlockSpec(memory_space=pltpu.SMEM),
            pl.BlockSpec(memory_space=pltpu.VMEM),
            pl.BlockSpec(memory_space=pltpu.VMEM),
            pl.BlockSpec(memory_space=pltpu.VMEM),
        ],
    )(w, pa.reshape(NP // 128, 128), pb.reshape(NP // 128, 128),
      res.reshape(NP // 128, 128))
    return y2.reshape(NP), r2.reshape(NP)


def kernel(x, edge_index, edge_probs, weights):
    src = edge_index[0].astype(jnp.int32)
    dst = edge_index[1].astype(jnp.int32)
    probs = edge_probs.astype(jnp.float32)

    pad = EPAD - N_EDGES
    src_p = jnp.pad(src, (0, pad)).reshape(N_TILES, N_CHUNKS, CHUNK_R, CHUNK_C)
    dst_p = jnp.pad(dst, (0, pad)).reshape(N_TILES, N_CHUNKS, CHUNK_R, CHUNK_C)
    prb_p = jnp.pad(probs, (0, pad)).reshape(N_TILES, N_CHUNKS, CHUNK_R, CHUNK_C)

    x0 = jnp.pad(x[:, 0], (0, NP - N_NODES))
    zeros_np = jnp.zeros((NP,), jnp.float32)
    w = weights.astype(jnp.float32)

    y, res = _combine(w[0:1], x0, zeros_np, zeros_np)
    for k in range(1, N_STEPS):
        pa, pb = _step(y, src_p, dst_p, prb_p)
        y, res = _combine(w[k:k + 1], pa, pb, res)
    return res[:N_NODES, None]


# speedup vs baseline: 421.6463x; 1.1221x over previous
import functools

import jax
import jax.numpy as jnp
from jax import lax
from jax.experimental import pallas as pl
from jax.experimental.pallas import tpu as pltpu
from jax.experimental.pallas import tpu_sc as plsc

N_NODES = 100000
N_EDGES = 6400000
N_STEPS = 5

NC = 2
NS = 16
N_TILES = NC * NS

SLICE = 6400
NP = NS * SLICE

CHUNK_R = 8
CHUNK_C = 128
CHUNK = CHUNK_R * CHUNK_C
N_CHUNKS = N_EDGES // CHUNK
N_UNIF = N_CHUNKS // N_TILES
N_EXTRA = N_CHUNKS - N_UNIF * N_TILES
NBUF = 4

_PIECES = [(0, 2048), (2048, 2048), (4096, 2048), (6144, 256)]

_mesh = plsc.VectorSubcoreMesh(core_axis_name="c", subcore_axis_name="s")
_params = pltpu.CompilerParams(needs_layout_passes=False)


def _step_body(x_in, src, dst, prb, pa_out, pb_out,
               x_buf, stage,
               src_v0, src_v1, src_v2, src_v3, dst_v0, dst_v1, dst_v2, dst_v3,
               prb_v0, prb_v1, prb_v2, prb_v3, val_v0, val_v1, val_v2, val_v3,
               sem_in0, sem_in1, sem_in2, sem_in3, sem_sc, acc):
    cid = lax.axis_index("c")
    sid = lax.axis_index("s")
    tid = cid * NS + sid
    base = sid * SLICE
    sems = [sem_in0, sem_in1, sem_in2, sem_in3]
    src_vs = [src_v0, src_v1, src_v2, src_v3]
    dst_vs = [dst_v0, dst_v1, dst_v2, dst_v3]
    prb_vs = [prb_v0, prb_v1, prb_v2, prb_v3]
    val_vs = [val_v0, val_v1, val_v2, val_v3]

    pltpu.sync_copy(x_in, x_buf)

    zv = jnp.zeros((16,), jnp.float32)

    def zero_body(j, _):
        stage[pl.ds(j * 16, 16)] = zv
        return _

    lax.fori_loop(0, 128, zero_body, 0)
    for off, sz in _PIECES:
        pltpu.sync_copy(stage.at[pl.ds(0, sz)], acc.at[pl.ds(base + off, sz)])
    plsc.subcore_barrier()

    def issue(j, b):
        g = tid + N_TILES * j
        pltpu.async_copy(src.at[g], src_vs[b], sems[b])
        pltpu.async_copy(dst.at[g], dst_vs[b], sems[b])
        pltpu.async_copy(prb.at[g], prb_vs[b], sems[b])

    def wait_in(j, b):
        g = tid + N_TILES * j
        pltpu.make_async_copy(src.at[g], src_vs[b], sems[b]).wait()
        pltpu.make_async_copy(dst.at[g], dst_vs[b], sems[b]).wait()
        pltpu.make_async_copy(prb.at[g], prb_vs[b], sems[b]).wait()

    def gather(b):
        for r in range(CHUNK_R):
            for j in range(CHUNK_C // 16):
                idx = src_vs[b][r, pl.ds(j * 16, 16)]
                v = (plsc.load_gather(x_buf, [idx])
                     * prb_vs[b][r, pl.ds(j * 16, 16)])
                val_vs[b][r, pl.ds(j * 16, 16)] = v

    def fire(b):
        for r in range(CHUNK_R):
            pltpu.async_copy(val_vs[b].at[r], acc.at[dst_vs[b].at[r]], sem_sc,
                             add=True)

    def drain(b):
        pltpu.make_async_copy(src.at[tid], dst_vs[b], sem_sc).wait()

    issue(0, 0)
    issue(1, 1)

    def loop_body(t, carry):
        for u in range(NBUF):
            j = t * NBUF + u
            wait_in(j, u)
            gather(u)
            if u <= 1:
                @pl.when(t >= 1)
                def _():
                    drain((u + 2) % NBUF)
            else:
                drain(u - 2)
            issue(j + 2, (u + 2) % NBUF)
            fire(u)
        return carry

    n_main = (N_UNIF - 3) // NBUF
    lax.fori_loop(0, n_main, loop_body, 0)
    for j in range(n_main * NBUF, N_UNIF):
        u = j % NBUF
        wait_in(j, u)
        gather(u)
        drain((u + 2) % NBUF)
        if j + 2 < N_UNIF:
            issue(j + 2, (j + 2) % NBUF)
        fire(u)
    drain((N_UNIF - 2) % NBUF)
    drain((N_UNIF - 1) % NBUF)

    @pl.when(tid < N_EXTRA)
    def _():
        g = N_UNIF * N_TILES + tid
        pltpu.sync_copy(src.at[g], src_vs[0])
        pltpu.sync_copy(dst.at[g], dst_vs[0])
        pltpu.sync_copy(prb.at[g], prb_vs[0])
        gather(0)
        for r in range(CHUNK_R):
            pltpu.sync_copy(val_vs[0].at[r], acc.at[dst_vs[0].at[r]], add=True)

    plsc.subcore_barrier()

    for off, sz in _PIECES:
        pltpu.sync_copy(acc.at[pl.ds(base + off, sz)], stage.at[pl.ds(0, sz)])

        @pl.when(cid == 0)
        def _():
            pltpu.sync_copy(stage.at[pl.ds(0, sz)],
                            pa_out.at[pl.ds(base + off, sz)])

        @pl.when(cid == 1)
        def _():
            pltpu.sync_copy(stage.at[pl.ds(0, sz)],
                            pb_out.at[pl.ds(base + off, sz)])


_step = functools.partial(
    pl.kernel,
    out_type=(jax.ShapeDtypeStruct((NP,), jnp.float32),
              jax.ShapeDtypeStruct((NP,), jnp.float32)),
    mesh=_mesh,
    scratch_types=[
        pltpu.VMEM((NP,), jnp.float32),
        pltpu.VMEM((2048,), jnp.float32),
        *[pltpu.VMEM((CHUNK_R, CHUNK_C), jnp.int32)
          for _ in range(2 * NBUF)],
        *[pltpu.VMEM((CHUNK_R, CHUNK_C), jnp.float32)
          for _ in range(2 * NBUF)],
        *[pltpu.SemaphoreType.DMA for _ in range(NBUF)],
        pltpu.SemaphoreType.DMA,
        pltpu.VMEM_SHARED((NP,), jnp.float32),
    ],
    compiler_params=_params,
)(_step_body)


def _combine_body(w_ref, pa_ref, pb_ref, res_ref, y_out, res_out):
    y = pa_ref[...] + pb_ref[...]
    y_out[...] = y
    res_out[...] = res_ref[...] + w_ref[0] * y


def _combine(w, pa, pb, res):
    y2, r2 = pl.pallas_call(
        _combine_body,
        out_shape=(jax.ShapeDtypeStruct((NP // 128, 128), jnp.float32),
                   jax.ShapeDtypeStruct((NP // 128, 128), jnp.float32)),
        in_specs=[
            pl.BlockSpec(memory_space=pltpu.SMEM),
            pl.BlockSpec(memory_space=pltpu.VMEM),
            pl.BlockSpec(memory_space=pltpu.VMEM),
            pl.BlockSpec(memory_space=pltpu.VMEM),
        ],
    )(w, pa.reshape(NP // 128, 128), pb.reshape(NP // 128, 128),
      res.reshape(NP // 128, 128))
    return y2.reshape(NP), r2.reshape(NP)


def kernel(x, edge_index, edge_probs, weights):
    src_p = edge_index[0].astype(jnp.int32).reshape(N_CHUNKS, CHUNK_R, CHUNK_C)
    dst_p = edge_index[1].astype(jnp.int32).reshape(N_CHUNKS, CHUNK_R, CHUNK_C)
    prb_p = edge_probs.astype(jnp.float32).reshape(N_CHUNKS, CHUNK_R, CHUNK_C)

    x0 = jnp.pad(x[:, 0], (0, NP - N_NODES))
    zeros_np = jnp.zeros((NP,), jnp.float32)
    w = weights.astype(jnp.float32)

    y, res = _combine(w[0:1], x0, zeros_np, zeros_np)
    for k in range(1, N_STEPS):
        pa, pb = _step(y, src_p, dst_p, prb_p)
        y, res = _combine(w[k:k + 1], pa, pb, res)
    return res[:N_NODES, None]


# speedup vs baseline: 438.5402x; 1.0401x over previous
import functools

import jax
import jax.numpy as jnp
from jax import lax
from jax.experimental import pallas as pl
from jax.experimental.pallas import tpu as pltpu
from jax.experimental.pallas import tpu_sc as plsc

N_NODES = 100000
N_EDGES = 6400000
N_STEPS = 5

NC = 2
NS = 16
N_TILES = NC * NS

SLICE = 6400
NP = NS * SLICE

CHUNK_R = 8
CHUNK_C = 128
CHUNK = CHUNK_R * CHUNK_C
N_CHUNKS = N_EDGES // CHUNK
N_UNIF = N_CHUNKS // N_TILES
N_EXTRA = N_CHUNKS - N_UNIF * N_TILES
NBUF = 4

_PIECES = [(0, 2048), (2048, 2048), (4096, 2048), (6144, 256)]

_mesh = plsc.VectorSubcoreMesh(core_axis_name="c", subcore_axis_name="s")
_params = pltpu.CompilerParams(needs_layout_passes=False)


def _step_body(x_in, ei, prb, pa_out, pb_out,
               x_buf, stage,
               src_v0, src_v1, src_v2, src_v3, dst_v0, dst_v1, dst_v2, dst_v3,
               prb_v0, prb_v1, prb_v2, prb_v3, val_v0, val_v1, val_v2, val_v3,
               sem_in0, sem_in1, sem_in2, sem_in3, sem_sc, acc):
    cid = lax.axis_index("c")
    sid = lax.axis_index("s")
    tid = cid * NS + sid
    base = sid * SLICE
    sems = [sem_in0, sem_in1, sem_in2, sem_in3]
    src_vs = [src_v0, src_v1, src_v2, src_v3]
    dst_vs = [dst_v0, dst_v1, dst_v2, dst_v3]
    prb_vs = [prb_v0, prb_v1, prb_v2, prb_v3]
    val_vs = [val_v0, val_v1, val_v2, val_v3]

    pltpu.sync_copy(x_in, x_buf)

    zv = jnp.zeros((16,), jnp.float32)

    def zero_body(j, _):
        stage[pl.ds(j * 16, 16)] = zv
        return _

    lax.fori_loop(0, 128, zero_body, 0)
    for off, sz in _PIECES:
        pltpu.sync_copy(stage.at[pl.ds(0, sz)], acc.at[pl.ds(base + off, sz)])
    plsc.subcore_barrier()

    def issue(j, b):
        g = tid + N_TILES * j
        pltpu.async_copy(ei.at[0, g], src_vs[b], sems[b])
        pltpu.async_copy(ei.at[1, g], dst_vs[b], sems[b])
        pltpu.async_copy(prb.at[g], prb_vs[b], sems[b])

    def wait_in(j, b):
        g = tid + N_TILES * j
        pltpu.make_async_copy(ei.at[0, g], src_vs[b], sems[b]).wait()
        pltpu.make_async_copy(ei.at[1, g], dst_vs[b], sems[b]).wait()
        pltpu.make_async_copy(prb.at[g], prb_vs[b], sems[b]).wait()

    def gather(b):
        for r in range(CHUNK_R):
            for j in range(CHUNK_C // 16):
                idx = src_vs[b][r, pl.ds(j * 16, 16)]
                v = (plsc.load_gather(x_buf, [idx])
                     * prb_vs[b][r, pl.ds(j * 16, 16)])
                val_vs[b][r, pl.ds(j * 16, 16)] = v

    def fire(b):
        for r in range(CHUNK_R):
            pltpu.async_copy(val_vs[b].at[r], acc.at[dst_vs[b].at[r]], sem_sc,
                             add=True)

    def drain(b):
        pltpu.make_async_copy(ei.at[0, tid], dst_vs[b], sem_sc).wait()

    issue(0, 0)
    issue(1, 1)

    def loop_body(t, carry):
        for u in range(NBUF):
            j = t * NBUF + u
            wait_in(j, u)
            gather(u)
            if u <= 1:
                @pl.when(t >= 1)
                def _():
                    drain((u + 2) % NBUF)
            else:
                drain(u - 2)
            issue(j + 2, (u + 2) % NBUF)
            fire(u)
        return carry

    n_main = (N_UNIF - 3) // NBUF
    lax.fori_loop(0, n_main, loop_body, 0)
    for j in range(n_main * NBUF, N_UNIF):
        u = j % NBUF
        wait_in(j, u)
        gather(u)
        drain((u + 2) % NBUF)
        if j + 2 < N_UNIF:
            issue(j + 2, (j + 2) % NBUF)
        fire(u)
    drain((N_UNIF - 2) % NBUF)
    drain((N_UNIF - 1) % NBUF)

    @pl.when(tid < N_EXTRA)
    def _():
        g = N_UNIF * N_TILES + tid
        pltpu.sync_copy(ei.at[0, g], src_vs[0])
        pltpu.sync_copy(ei.at[1, g], dst_vs[0])
        pltpu.sync_copy(prb.at[g], prb_vs[0])
        gather(0)
        for r in range(CHUNK_R):
            pltpu.sync_copy(val_vs[0].at[r], acc.at[dst_vs[0].at[r]], add=True)

    plsc.subcore_barrier()

    for off, sz in _PIECES:
        pltpu.sync_copy(acc.at[pl.ds(base + off, sz)], stage.at[pl.ds(0, sz)])

        @pl.when(cid == 0)
        def _():
            pltpu.sync_copy(stage.at[pl.ds(0, sz)],
                            pa_out.at[pl.ds(base + off, sz)])

        @pl.when(cid == 1)
        def _():
            pltpu.sync_copy(stage.at[pl.ds(0, sz)],
                            pb_out.at[pl.ds(base + off, sz)])


_step = functools.partial(
    pl.kernel,
    out_type=(jax.ShapeDtypeStruct((NP,), jnp.float32),
              jax.ShapeDtypeStruct((NP,), jnp.float32)),
    mesh=_mesh,
    scratch_types=[
        pltpu.VMEM((NP,), jnp.float32),
        pltpu.VMEM((2048,), jnp.float32),
        *[pltpu.VMEM((CHUNK_R, CHUNK_C), jnp.int32)
          for _ in range(2 * NBUF)],
        *[pltpu.VMEM((CHUNK_R, CHUNK_C), jnp.float32)
          for _ in range(2 * NBUF)],
        *[pltpu.SemaphoreType.DMA for _ in range(NBUF)],
        pltpu.SemaphoreType.DMA,
        pltpu.VMEM_SHARED((NP,), jnp.float32),
    ],
    compiler_params=_params,
)(_step_body)


def _combine_body(w_ref, pa_ref, pb_ref, res_ref, y_out, res_out):
    y = pa_ref[...] + pb_ref[...]
    y_out[...] = y
    res_out[...] = res_ref[...] + w_ref[0] * y


def _combine(w, pa, pb, res):
    y2, r2 = pl.pallas_call(
        _combine_body,
        out_shape=(jax.ShapeDtypeStruct((NP // 128, 128), jnp.float32),
                   jax.ShapeDtypeStruct((NP // 128, 128), jnp.float32)),
        in_specs=[
            pl.BlockSpec(memory_space=pltpu.SMEM),
            pl.BlockSpec(memory_space=pltpu.VMEM),
            pl.BlockSpec(memory_space=pltpu.VMEM),
            pl.BlockSpec(memory_space=pltpu.VMEM),
        ],
    )(w, pa.reshape(NP // 128, 128), pb.reshape(NP // 128, 128),
      res.reshape(NP // 128, 128))
    return y2.reshape(NP), r2.reshape(NP)


def _fin_body(w_ref, pa_ref, pb_ref, res_ref, x0_ref, res_out):
    res_out[...] = (res_ref[...] + w_ref[0] * (pa_ref[...] + pb_ref[...])
                    + w_ref[1] * x0_ref[...])


def _fin(w2, pa, pb, res, x0):
    r2 = pl.pallas_call(
        _fin_body,
        out_shape=jax.ShapeDtypeStruct((NP // 128, 128), jnp.float32),
        in_specs=[
            pl.BlockSpec(memory_space=pltpu.SMEM),
            pl.BlockSpec(memory_space=pltpu.VMEM),
            pl.BlockSpec(memory_space=pltpu.VMEM),
            pl.BlockSpec(memory_space=pltpu.VMEM),
            pl.BlockSpec(memory_space=pltpu.VMEM),
        ],
    )(w2, pa.reshape(NP // 128, 128), pb.reshape(NP // 128, 128),
      res.reshape(NP // 128, 128), x0.reshape(NP // 128, 128))
    return r2.reshape(NP)


def kernel(x, edge_index, edge_probs, weights):
    ei = edge_index.astype(jnp.int32).reshape(2, N_CHUNKS, CHUNK_R, CHUNK_C)
    prb_p = edge_probs.astype(jnp.float32).reshape(N_CHUNKS, CHUNK_R, CHUNK_C)

    x0 = jnp.pad(x[:, 0], (0, NP - N_NODES))
    zeros_np = jnp.zeros((NP,), jnp.float32)
    w = weights.astype(jnp.float32)

    y, res = x0, zeros_np
    for k in range(1, N_STEPS - 1):
        pa, pb = _step(y, ei, prb_p)
        y, res = _combine(w[k:k + 1], pa, pb, res)
    pa, pb = _step(y, ei, prb_p)
    res = _fin(jnp.stack([w[N_STEPS - 1], w[0]]), pa, pb, res, x0)
    return res[:N_NODES, None]
